# Initial kernel scaffold; baseline (speedup 1.0000x reference)
import jax
import jax.numpy as jnp
from jax.experimental import pallas as pl


def kernel(z, edge_index, W0, b0, W1, b1, W2, b2, Wr1, br1, Wr2, br2, Wt1, bt1, Wt2, bt2):
    raise NotImplementedError("write your pallas kernel here")


# speedup vs baseline: 2.1996x; 2.1996x over previous
import functools

import jax
import jax.numpy as jnp
from jax.experimental import pallas as pl
from jax.experimental.pallas import tpu as pltpu

_RADIUS = 1.0
_EPS = 1.0 + 1e-7


def _edge_body(ts_ref, td_ref, wcat_ref, bcat_ref, w2_ref, tail_ref, out_ref):
    ts = ts_ref[...]
    td = td_ref[...]
    gs, vs = ts[:, :32], ts[:, 32:48]
    gd, vd = td[:, :32], td[:, 32:48]
    z0s = jnp.sqrt(_RADIUS**2 + jnp.sum(vs * vs, axis=1))
    z0d = jnp.sqrt(_RADIUS**2 + jnp.sum(vd * vd, axis=1))
    inner = -z0s * z0d + jnp.sum(vs * vd, axis=1)
    arg = jnp.maximum(-inner / (_RADIUS**2), _EPS)
    dist = -_RADIUS * jnp.log(arg + jnp.sqrt(arg * arg - 1.0))
    wcat = wcat_ref[...]
    h = (
        jnp.dot(gs, wcat[:32], preferred_element_type=jnp.float32)
        + jnp.dot(gd, wcat[32:], preferred_element_type=jnp.float32)
        + bcat_ref[...]
    )
    h = jnp.where(h >= 0.0, h, 0.2 * h)
    rt = h * w2_ref[...]
    r = jnp.sum(rt[:, :64], axis=1) + tail_ref[0, 0]
    t = jnp.sum(rt[:, 64:], axis=1) + tail_ref[0, 1]
    out_ref[0, 0, :] = jax.nn.sigmoid((dist - r) / t)


def _edge_score(ts, td, wcat, bcat, w2, tail):
    E = ts.shape[0]
    bE = 10000
    grid = E // bE
    return pl.pallas_call(
        _edge_body,
        grid=(grid,),
        in_specs=[
            pl.BlockSpec((bE, 48), lambda i: (i, 0)),
            pl.BlockSpec((bE, 48), lambda i: (i, 0)),
            pl.BlockSpec((64, 128), lambda i: (0, 0)),
            pl.BlockSpec((1, 128), lambda i: (0, 0)),
            pl.BlockSpec((1, 128), lambda i: (0, 0)),
            pl.BlockSpec((1, 128), lambda i: (0, 0)),
        ],
        out_specs=pl.BlockSpec((1, 1, bE), lambda i: (i, 0, 0)),
        out_shape=jax.ShapeDtypeStruct((grid, 1, bE), jnp.float32),
    )(ts, td, wcat, bcat, w2, tail).reshape(E)


def _inverse_exp_map_mu0(x):
    x0 = x[:, :1]
    alpha = jnp.maximum(x0 / _RADIUS, _EPS)
    coef = jnp.arccosh(alpha) / jnp.sqrt(alpha**2 - 1.0)
    proj = jnp.concatenate([x0 - alpha * _RADIUS, x[:, 1:]], axis=1)
    return coef * proj


def kernel(z, edge_index, W0, b0, W1, b1, W2, b2, Wr1, br1, Wr2, br2, Wt1, bt1, Wt2, bt2):
    N = z.shape[0]
    src, dst = edge_index[0], edge_index[1]
    deg = 1.0 + jax.ops.segment_sum(jnp.ones_like(src, jnp.float32), dst, num_segments=N)
    dinv = jax.lax.rsqrt(deg)[:, None]

    z_mu0 = _inverse_exp_map_mu0(z)
    h = z_mu0
    for i, (W, b) in enumerate([(W0, b0), (W1, b1), (W2, b2)]):
        if i > 0:
            h = jax.nn.relu(h)
        X = (h @ W) * dinv
        S = jax.ops.segment_sum(X[src], dst, num_segments=N)
        h = dinv * (S + X) + b
    g = h

    T = jnp.concatenate([g, z[:, 1:]], axis=1)
    ts = T[src]
    td = T[dst]
    wcat = jnp.concatenate([Wr1, Wt1], axis=1)
    bcat = jnp.concatenate([br1, bt1])[None, :]
    w2 = jnp.concatenate([Wr2[:, 0], Wt2[:, 0]])[None, :]
    tail = jnp.zeros((1, 128), jnp.float32).at[0, 0].set(br2[0]).at[0, 1].set(bt2[0])
    return _edge_score(ts, td, wcat, bcat, w2, tail)
